# baseline (device time: 131918 ns/iter reference)
import jax
import jax.numpy as jnp
from jax import lax
from jax.experimental import pallas as pl
from jax.experimental.pallas import tpu as pltpu

N_EXP_LOCAL = 4
N_EXP = 8
CAP = 320
F_TILE = 256


def kernel(x, assign, W1, W2):
    T, D = x.shape
    E, _, F = W1.shape
    n_ft = F // F_TILE

    my_x = lax.axis_index("x")
    base = N_EXP_LOCAL * my_x

    l = jnp.mod(assign - base, N_EXP)
    oh = (l[:, None] == jnp.arange(N_EXP)[None, :]).astype(jnp.int32)
    rank = jnp.take_along_axis(jnp.cumsum(oh, axis=0), l[:, None], axis=1)[:, 0] - 1
    slots = l * CAP + jnp.minimum(rank, CAP - 1)
    tok4slot = (
        jnp.full((N_EXP * CAP,), -1, jnp.int32)
        .at[slots]
        .set(jnp.arange(T, dtype=jnp.int32))
    )
    xb = x.astype(jnp.bfloat16)

    def body(xb_ref, sl_ref, t4s_ref, w1_ref, w2_ref, out_ref,
             xs_ref, xp_ref, yr_ref, ys_ref, accm, accp,
             s_sems, rx_sems, ry_sems):
        e = pl.program_id(0)
        ft = pl.program_id(1)
        mx = lax.axis_index("x")
        peer = (1 - mx, lax.axis_index("y"), lax.axis_index("z"))

        def x_rdma(j):
            return pltpu.make_async_remote_copy(
                src_ref=ys_ref.at[j], dst_ref=xp_ref.at[j],
                send_sem=s_sems.at[j], recv_sem=rx_sems.at[j],
                device_id=peer, device_id_type=pl.DeviceIdType.MESH,
            )

        def y_rdma(j):
            return pltpu.make_async_remote_copy(
                src_ref=ys_ref.at[j], dst_ref=yr_ref.at[j],
                send_sem=s_sems.at[N_EXP_LOCAL + j], recv_sem=ry_sems.at[j],
                device_id=peer, device_id_type=pl.DeviceIdType.MESH,
            )

        GCH = 512

        def gather_row(row):
            vals = t4s_ref[pl.ds(row * CAP, CAP), :]
            acc = jnp.zeros((CAP, D), jnp.float32)
            for tc in range(T // GCH):
                p = (vals == tc * GCH
                     + lax.broadcasted_iota(jnp.int32, (CAP, GCH), 1)
                     ).astype(jnp.bfloat16)
                acc = acc + jnp.dot(
                    p, xb_ref[tc * GCH:(tc + 1) * GCH, :],
                    preferred_element_type=jnp.float32,
                )
            return acc.astype(jnp.bfloat16)

        def unperm_add(row_start, yv):
            for tc in range(T // GCH):
                u = (sl_ref[tc * GCH:(tc + 1) * GCH, :] == row_start
                     + lax.broadcasted_iota(jnp.int32, (GCH, CAP), 1)
                     ).astype(jnp.bfloat16)
                sl = slice(tc * GCH, (tc + 1) * GCH)
                out_ref[sl, :] = out_ref[sl, :] + jnp.dot(
                    u, yv, preferred_element_type=jnp.float32
                )

        @pl.when(jnp.logical_and(e == 0, ft == 0))
        def _():
            barrier_sem = pltpu.get_barrier_semaphore()
            pl.semaphore_signal(
                barrier_sem, inc=1,
                device_id=peer, device_id_type=pl.DeviceIdType.MESH,
            )
            pl.semaphore_wait(barrier_sem, 1)
            out_ref[...] = jnp.zeros_like(out_ref)
            for j in range(N_EXP_LOCAL):
                ys_ref[j] = gather_row(N_EXP_LOCAL + j)
                x_rdma(j).start()

        w1 = w1_ref[...].astype(jnp.bfloat16)
        w2 = w2_ref[...].astype(jnp.bfloat16)

        @pl.when(ft == 0)
        def _():
            xs_ref[pl.ds(e, 1)] = gather_row(e)[None]

        xm = xs_ref[pl.ds(e, 1)][0]
        hm = jnp.maximum(
            jnp.dot(xm, w1, preferred_element_type=jnp.float32), 0.0
        ).astype(jnp.bfloat16)
        ym = jnp.dot(hm, w2, preferred_element_type=jnp.float32)

        @pl.when(ft == 0)
        def _():
            accm[...] = ym
            x_rdma(e).wait_recv()

        @pl.when(ft != 0)
        def _():
            accm[...] = accm[...] + ym

        xpv = xp_ref[pl.ds(e, 1)][0]
        hp = jnp.maximum(
            jnp.dot(xpv, w1, preferred_element_type=jnp.float32), 0.0
        ).astype(jnp.bfloat16)
        yp = jnp.dot(hp, w2, preferred_element_type=jnp.float32)

        @pl.when(ft == 0)
        def _():
            accp[...] = yp

        @pl.when(ft != 0)
        def _():
            accp[...] = accp[...] + yp

        @pl.when(ft == n_ft - 1)
        def _():
            unperm_add(e * CAP, accm[...].astype(jnp.bfloat16))
            x_rdma(e).wait_send()
            ys_ref[pl.ds(e, 1)] = accp[...].astype(jnp.bfloat16)[None]
            y_rdma(e).start()

        @pl.when(jnp.logical_and(e == E - 1, ft == n_ft - 1))
        def _():
            for j in range(N_EXP_LOCAL):
                y_rdma(j).wait_recv()
                unperm_add((N_EXP_LOCAL + j) * CAP, yr_ref[j])
            for j in range(N_EXP_LOCAL):
                y_rdma(j).wait_send()

    return pl.pallas_call(
        body,
        grid=(E, n_ft),
        out_shape=jax.ShapeDtypeStruct((T, D), jnp.float32),
        in_specs=[
            pl.BlockSpec((T, D), lambda e, ft: (0, 0)),
            pl.BlockSpec((T, 1), lambda e, ft: (0, 0)),
            pl.BlockSpec((N_EXP * CAP, 1), lambda e, ft: (0, 0)),
            pl.BlockSpec((None, D, F_TILE), lambda e, ft: (e, 0, ft)),
            pl.BlockSpec((None, F_TILE, D), lambda e, ft: (e, ft, 0)),
        ],
        out_specs=pl.BlockSpec((T, D), lambda e, ft: (0, 0)),
        scratch_shapes=[
            pltpu.VMEM((N_EXP_LOCAL, CAP, D), jnp.bfloat16),
            pltpu.VMEM((N_EXP_LOCAL, CAP, D), jnp.bfloat16),
            pltpu.VMEM((N_EXP_LOCAL, CAP, D), jnp.bfloat16),
            pltpu.VMEM((N_EXP_LOCAL, CAP, D), jnp.bfloat16),
            pltpu.VMEM((CAP, D), jnp.float32),
            pltpu.VMEM((CAP, D), jnp.float32),
            pltpu.SemaphoreType.DMA((2 * N_EXP_LOCAL,)),
            pltpu.SemaphoreType.DMA((N_EXP_LOCAL,)),
            pltpu.SemaphoreType.DMA((N_EXP_LOCAL,)),
        ],
        compiler_params=pltpu.CompilerParams(collective_id=0),
    )(xb, slots.reshape(T, 1), tok4slot.reshape(N_EXP * CAP, 1), W1, W2)


# device time: 103166 ns/iter; 1.2787x vs baseline; 1.2787x over previous
import jax
import jax.numpy as jnp
from jax import lax
from jax.experimental import pallas as pl
from jax.experimental.pallas import tpu as pltpu

N_EXP_LOCAL = 4
N_EXP = 8
CAP = 320
F_TILE = 256
GCH = 512


def kernel(x, assign, W1, W2):
    T, D = x.shape
    E, _, F = W1.shape
    n_ft = F // F_TILE

    def body(xb_ref, a_ref, w1_ref, w2_ref, out_ref,
             sl_ref, xs_ref, xp_ref, yr_ref, ys_ref, accm, accp,
             s_sems, rx_sems, ry_sems):
        e = pl.program_id(0)
        ft = pl.program_id(1)
        mx = lax.axis_index("x")
        peer = (1 - mx, lax.axis_index("y"), lax.axis_index("z"))
        base = N_EXP_LOCAL * mx

        def x_rdma(j):
            return pltpu.make_async_remote_copy(
                src_ref=ys_ref.at[j], dst_ref=xp_ref.at[j],
                send_sem=s_sems.at[j], recv_sem=rx_sems.at[j],
                device_id=peer, device_id_type=pl.DeviceIdType.MESH,
            )

        def y_rdma(j):
            return pltpu.make_async_remote_copy(
                src_ref=ys_ref.at[j], dst_ref=yr_ref.at[j],
                send_sem=s_sems.at[N_EXP_LOCAL + j], recv_sem=ry_sems.at[j],
                device_id=peer, device_id_type=pl.DeviceIdType.MESH,
            )

        def u_chunk(tc, row_start):
            return (
                sl_ref[tc * GCH:(tc + 1) * GCH, :] == row_start
                + lax.broadcasted_iota(jnp.int32, (GCH, CAP), 1)
            ).astype(jnp.bfloat16)

        def gather_row(row):
            acc = jnp.zeros((CAP, D), jnp.float32)
            for tc in range(T // GCH):
                acc = acc + lax.dot_general(
                    u_chunk(tc, row * CAP),
                    xb_ref[tc * GCH:(tc + 1) * GCH, :],
                    (((0,), (0,)), ((), ())),
                    preferred_element_type=jnp.float32,
                )
            return acc.astype(jnp.bfloat16)

        def unperm_add(row_start, yv):
            for tc in range(T // GCH):
                sl = slice(tc * GCH, (tc + 1) * GCH)
                out_ref[sl, :] = out_ref[sl, :] + jnp.dot(
                    u_chunk(tc, row_start), yv,
                    preferred_element_type=jnp.float32,
                )

        @pl.when(jnp.logical_and(e == 0, ft == 0))
        def _():
            barrier_sem = pltpu.get_barrier_semaphore()
            pl.semaphore_signal(
                barrier_sem, inc=1,
                device_id=peer, device_id_type=pl.DeviceIdType.MESH,
            )
            pl.semaphore_wait(barrier_sem, 1)
            out_ref[...] = jnp.zeros_like(out_ref)

            lv = jnp.mod(a_ref[...] - base, N_EXP)
            tri = (
                lax.broadcasted_iota(jnp.int32, (GCH, GCH), 0)
                >= lax.broadcasted_iota(jnp.int32, (GCH, GCH), 1)
            ).astype(jnp.bfloat16)
            off = jnp.zeros((1, N_EXP), jnp.float32)
            for tc in range(T // GCH):
                sl = slice(tc * GCH, (tc + 1) * GCH)
                lc = lv[sl]
                ohc = (
                    lc == lax.broadcasted_iota(jnp.int32, (GCH, N_EXP), 1)
                ).astype(jnp.bfloat16)
                cc = jnp.dot(tri, ohc, preferred_element_type=jnp.float32) + off
                ohf = ohc.astype(jnp.float32)
                rank = jnp.sum(cc * ohf, axis=1, keepdims=True) - 1.0
                off = off + jnp.sum(ohf, axis=0, keepdims=True)
                sl_ref[sl, :] = lc * CAP + jnp.minimum(
                    rank.astype(jnp.int32), CAP - 1
                )

            for j in range(N_EXP_LOCAL):
                ys_ref[j] = gather_row(N_EXP_LOCAL + j)
                x_rdma(j).start()

        @pl.when(ft == 0)
        def _():
            xs_ref[pl.ds(e, 1)] = gather_row(e)[None]
            x_rdma(e).wait_recv()

        w1 = w1_ref[...].astype(jnp.bfloat16)
        w2 = w2_ref[...].astype(jnp.bfloat16)

        xm = xs_ref[pl.ds(e, 1)][0]
        hm = jnp.maximum(
            jnp.dot(xm, w1, preferred_element_type=jnp.float32), 0.0
        ).astype(jnp.bfloat16)
        ym = jnp.dot(hm, w2, preferred_element_type=jnp.float32)

        xpv = xp_ref[pl.ds(e, 1)][0]
        hp = jnp.maximum(
            jnp.dot(xpv, w1, preferred_element_type=jnp.float32), 0.0
        ).astype(jnp.bfloat16)
        yp = jnp.dot(hp, w2, preferred_element_type=jnp.float32)

        @pl.when(ft == 0)
        def _():
            accm[...] = ym
            accp[...] = yp

        @pl.when(ft != 0)
        def _():
            accm[...] = accm[...] + ym
            accp[...] = accp[...] + yp

        @pl.when(ft == n_ft - 1)
        def _():
            unperm_add(e * CAP, accm[...].astype(jnp.bfloat16))
            x_rdma(e).wait_send()
            ys_ref[pl.ds(e, 1)] = accp[...].astype(jnp.bfloat16)[None]
            y_rdma(e).start()

        @pl.when(jnp.logical_and(e == E - 1, ft == n_ft - 1))
        def _():
            for j in range(N_EXP_LOCAL):
                y_rdma(j).wait_recv()
                unperm_add((N_EXP_LOCAL + j) * CAP, yr_ref[j])
            for j in range(N_EXP_LOCAL):
                y_rdma(j).wait_send()

    return pl.pallas_call(
        body,
        grid=(E, n_ft),
        out_shape=jax.ShapeDtypeStruct((T, D), jnp.float32),
        in_specs=[
            pl.BlockSpec((T, D), lambda e, ft: (0, 0)),
            pl.BlockSpec((T, 1), lambda e, ft: (0, 0)),
            pl.BlockSpec((None, D, F_TILE), lambda e, ft: (e, 0, ft)),
            pl.BlockSpec((None, F_TILE, D), lambda e, ft: (e, ft, 0)),
        ],
        out_specs=pl.BlockSpec((T, D), lambda e, ft: (0, 0)),
        scratch_shapes=[
            pltpu.VMEM((T, 1), jnp.int32),
            pltpu.VMEM((N_EXP_LOCAL, CAP, D), jnp.bfloat16),
            pltpu.VMEM((N_EXP_LOCAL, CAP, D), jnp.bfloat16),
            pltpu.VMEM((N_EXP_LOCAL, CAP, D), jnp.bfloat16),
            pltpu.VMEM((N_EXP_LOCAL, CAP, D), jnp.bfloat16),
            pltpu.VMEM((CAP, D), jnp.float32),
            pltpu.VMEM((CAP, D), jnp.float32),
            pltpu.SemaphoreType.DMA((2 * N_EXP_LOCAL,)),
            pltpu.SemaphoreType.DMA((N_EXP_LOCAL,)),
            pltpu.SemaphoreType.DMA((N_EXP_LOCAL,)),
        ],
        compiler_params=pltpu.CompilerParams(collective_id=0),
    )(x.astype(jnp.bfloat16), assign.reshape(T, 1), W1, W2)
